# baseline (speedup 1.0000x reference)
import functools

import jax
import jax.numpy as jnp
from jax.experimental import pallas as pl
from jax.experimental.pallas import tpu as pltpu
from jax.experimental.pallas import tpu_sc as plsc

_SEC0 = 4096

_BM = 256
_BN = 512


def _rb(v):
    return v.astype(jnp.bfloat16).astype(jnp.float32)


def _argmin_body(lhs_ref, rhs_ref, ssq_ref, c_ref, ind_ref, *, n_total):
    nb = n_total // _BN
    ssq = ssq_ref[...]
    lhs = lhs_ref[...]

    def n_step(ni, carry):
        bv, bi = carry
        rhs = rhs_ref[pl.ds(ni * _BN, _BN), :]
        s = jax.lax.dot_general(
            lhs,
            rhs,
            dimension_numbers=(((1,), (1,)), ((), ())),
            preferred_element_type=jnp.float32,
        )
        negd = -((ssq - s) + c_ref[:, pl.ds(ni * _BN, _BN)])
        tv = jnp.max(negd, axis=1, keepdims=True)
        iota = jax.lax.broadcasted_iota(jnp.int32, (_BM, _BN), 1)
        cand = jnp.where(negd == tv, iota, _BN)
        ti = jnp.min(cand, axis=1, keepdims=True) + ni * _BN
        take = tv > bv
        return jnp.where(take, tv, bv), jnp.where(take, ti, bi)

    neg_inf = jnp.full((_BM, 1), -jnp.inf, dtype=jnp.float32)
    zero_i = jnp.zeros((_BM, 1), dtype=jnp.int32)
    half = _SEC0 // _BN
    bv0, bi0 = jax.lax.fori_loop(0, half, n_step, (neg_inf, zero_i))
    bv1, bi1 = jax.lax.fori_loop(half, nb, n_step, (neg_inf, zero_i))

    acc_v = _rb(bv0)
    take = bv1 > acc_v
    acc_i = jnp.where(take, bi1, bi0)
    ind_ref[...] = acc_i


def _argmin(lhs, rhs, ssq, c_row):
    m_total, dim = lhs.shape
    n_total = rhs.shape[0]
    return pl.pallas_call(
        functools.partial(_argmin_body, n_total=n_total),
        grid=(m_total // _BM,),
        in_specs=[
            pl.BlockSpec((_BM, dim), lambda i: (i, 0)),
            pl.BlockSpec((n_total, dim), lambda i: (0, 0)),
            pl.BlockSpec((_BM, 1), lambda i: (i, 0)),
            pl.BlockSpec((1, n_total), lambda i: (0, 0)),
        ],
        out_specs=pl.BlockSpec((_BM, 1), lambda i: (i, 0)),
        out_shape=jax.ShapeDtypeStruct((m_total, 1), jnp.int32),
    )(lhs, rhs, ssq, c_row)


_GATHER_WINDOW = 128


def _sc_gather(renormed, ind_row):
    n_embed, dim = renormed.shape
    num_idx = ind_row.shape[1]
    mesh = plsc.VectorSubcoreMesh(core_axis_name="c", subcore_axis_name="s")

    @pl.kernel(
        out_type=jax.ShapeDtypeStruct((num_idx, dim), jnp.float32),
        mesh=mesh,
    )
    def gather_kernel(tbl_hbm, i_hbm, o_hbm):
        def body(i_vmem, o_vmem):
            pltpu.sync_copy(tbl_hbm.at[i_vmem.at[0]], o_vmem)

        pltpu.emit_pipeline(
            body,
            grid=(num_idx // _GATHER_WINDOW,),
            in_specs=[
                pl.BlockSpec((1, _GATHER_WINDOW), lambda i: (0, i)),
            ],
            out_specs=[
                pl.BlockSpec((_GATHER_WINDOW, dim), lambda i: (i, 0)),
            ],
            core_axis_name=("c", "s"),
            dimension_semantics=(pltpu.PARALLEL,),
        )(i_hbm, o_hbm)

    return gather_kernel(renormed, ind_row)


_DBLK = 2048


def _diff_body(q_ref, x_ref, o_ref, *, nblk, inv_count):
    i = pl.program_id(0)

    @pl.when(i == 0)
    def _():
        o_ref[...] = jnp.zeros_like(o_ref)

    d = q_ref[...] - x_ref[...]
    o_ref[...] += jnp.sum(d * d).reshape(1, 1)

    @pl.when(i == nblk - 1)
    def _():
        o_ref[...] = o_ref[...] * inv_count


def _diff(q, x):
    m_total, dim = q.shape
    nblk = m_total // _DBLK
    return pl.pallas_call(
        functools.partial(_diff_body, nblk=nblk, inv_count=1.0 / (m_total * dim)),
        grid=(nblk,),
        in_specs=[
            pl.BlockSpec((_DBLK, dim), lambda i: (i, 0)),
            pl.BlockSpec((_DBLK, dim), lambda i: (i, 0)),
        ],
        out_specs=pl.BlockSpec((1, 1), lambda i: (0, 0)),
        out_shape=jax.ShapeDtypeStruct((1, 1), jnp.float32),
    )(q, x)


def kernel(input, weight):
    b, t, dim = input.shape
    m_total = b * t
    flatten = input.reshape(m_total, dim)

    norms = jnp.sqrt(jnp.sum(weight * weight, axis=1))
    ebar = weight / norms[:, None]
    c = jnp.sum(ebar * ebar, axis=1)
    ssq = jnp.sum(flatten * flatten, axis=1)
    lhs = (2.0 * flatten).astype(jnp.bfloat16)
    rhs = ebar.astype(jnp.bfloat16)
    scale = jnp.where(norms > 1.0, 1.0 / (norms + 1e-7), 1.0)
    renormed = weight * scale[:, None]

    ind = _argmin(lhs, rhs, ssq.reshape(m_total, 1), c.reshape(1, -1))

    q = _sc_gather(renormed, ind.reshape(1, m_total))

    diff = _diff(q, flatten).reshape(())

    return (q.reshape(b, t, dim), diff, ind.reshape(b, t))


# speedup vs baseline: 3.1477x; 3.1477x over previous
import functools

import jax
import jax.numpy as jnp
from jax.experimental import pallas as pl
from jax.experimental.pallas import tpu as pltpu
from jax.experimental.pallas import tpu_sc as plsc

_SEC0 = 4096

_BM = 256
_BIG = 2**30


def _rb(v):
    return v.astype(jnp.bfloat16).astype(jnp.float32)


def _argmin_body(lhs_ref, rhs_ref, ssq_ref, c_ref, ind_ref, *, n_total):
    lhs = lhs_ref[...]
    rhs = rhs_ref[...]
    s = jax.lax.dot_general(
        rhs,
        lhs,
        dimension_numbers=(((1,), (1,)), ((), ())),
        preferred_element_type=jnp.float32,
    )
    d = (ssq_ref[...] - s) + c_ref[...]
    v3 = d.reshape(n_total // 8, 8, _BM)
    iota_s = jax.lax.broadcasted_iota(jnp.int32, (8, _BM), 0)

    def section(lo, hi):
        seg = v3[lo:hi]
        mv = jnp.min(seg, axis=0)
        mr = jnp.argmin(seg, axis=0).astype(jnp.int32)
        gidx = (mr + lo) * 8 + iota_s
        m = jnp.min(mv, axis=0, keepdims=True)
        cand = jnp.where(mv == m, gidx, _BIG)
        return m, jnp.min(cand, axis=0, keepdims=True)

    m0, i0 = section(0, _SEC0 // 8)
    m1, i1 = section(_SEC0 // 8, n_total // 8)
    take = m1 < _rb(m0)
    ind_ref[...] = jnp.where(take, i1, i0).reshape(1, 1, _BM)


def _argmin(lhs, rhs, ssq_row, c_col):
    m_total, dim = lhs.shape
    n_total = rhs.shape[0]
    nb = m_total // _BM
    out = pl.pallas_call(
        functools.partial(_argmin_body, n_total=n_total),
        grid=(nb,),
        in_specs=[
            pl.BlockSpec((_BM, dim), lambda i: (i, 0)),
            pl.BlockSpec((n_total, dim), lambda i: (0, 0)),
            pl.BlockSpec((1, _BM), lambda i: (0, i)),
            pl.BlockSpec((n_total, 1), lambda i: (0, 0)),
        ],
        out_specs=pl.BlockSpec((1, 1, _BM), lambda i: (i, 0, 0)),
        out_shape=jax.ShapeDtypeStruct((nb, 1, _BM), jnp.int32),
    )(lhs, rhs, ssq_row, c_col)
    return out.reshape(m_total)


_GATHER_WINDOW = 128


def _sc_gather(renormed, ind_row):
    n_embed, dim = renormed.shape
    num_idx = ind_row.shape[1]
    mesh = plsc.VectorSubcoreMesh(core_axis_name="c", subcore_axis_name="s")

    @pl.kernel(
        out_type=jax.ShapeDtypeStruct((num_idx, dim), jnp.float32),
        mesh=mesh,
    )
    def gather_kernel(tbl_hbm, i_hbm, o_hbm):
        def body(i_vmem, o_vmem):
            pltpu.sync_copy(tbl_hbm.at[i_vmem.at[0]], o_vmem)

        pltpu.emit_pipeline(
            body,
            grid=(num_idx // _GATHER_WINDOW,),
            in_specs=[
                pl.BlockSpec((1, _GATHER_WINDOW), lambda i: (0, i)),
            ],
            out_specs=[
                pl.BlockSpec((_GATHER_WINDOW, dim), lambda i: (i, 0)),
            ],
            core_axis_name=("c", "s"),
            dimension_semantics=(pltpu.PARALLEL,),
        )(i_hbm, o_hbm)

    return gather_kernel(renormed, ind_row)


_DBLK = 2048


def _diff_body(q_ref, x_ref, o_ref, *, nblk, inv_count):
    i = pl.program_id(0)

    @pl.when(i == 0)
    def _():
        o_ref[...] = jnp.zeros_like(o_ref)

    d = q_ref[...] - x_ref[...]
    o_ref[...] += jnp.sum(d * d).reshape(1, 1)

    @pl.when(i == nblk - 1)
    def _():
        o_ref[...] = o_ref[...] * inv_count


def _diff(q, x):
    m_total, dim = q.shape
    nblk = m_total // _DBLK
    return pl.pallas_call(
        functools.partial(_diff_body, nblk=nblk, inv_count=1.0 / (m_total * dim)),
        grid=(nblk,),
        in_specs=[
            pl.BlockSpec((_DBLK, dim), lambda i: (i, 0)),
            pl.BlockSpec((_DBLK, dim), lambda i: (i, 0)),
        ],
        out_specs=pl.BlockSpec((1, 1), lambda i: (0, 0)),
        out_shape=jax.ShapeDtypeStruct((1, 1), jnp.float32),
    )(q, x)


def kernel(input, weight):
    b, t, dim = input.shape
    m_total = b * t
    flatten = input.reshape(m_total, dim)

    norms = jnp.sqrt(jnp.sum(weight * weight, axis=1))
    ebar = weight / norms[:, None]
    c = jnp.sum(ebar * ebar, axis=1)
    ssq = jnp.sum(flatten * flatten, axis=1)
    lhs = (2.0 * flatten).astype(jnp.bfloat16)
    rhs = ebar.astype(jnp.bfloat16)
    scale = jnp.where(norms > 1.0, 1.0 / (norms + 1e-7), 1.0)
    renormed = weight * scale[:, None]

    ind = _argmin(lhs, rhs, ssq.reshape(1, m_total), c.reshape(-1, 1))

    q = _sc_gather(renormed, ind.reshape(1, m_total))

    diff = _diff(q, flatten).reshape(())

    return (q.reshape(b, t, dim), diff, ind.reshape(b, t))
